# baseline (speedup 1.0000x reference)
import jax
import jax.numpy as jnp
from jax.experimental import pallas as pl
from jax.experimental.pallas import tpu as pltpu

_F32 = jnp.float32

_NBL = 8
_T = 512
_S = _T + 8
_R = _NBL * _S
_RS = _R + 16


def _dot(a, b):
    return jnp.dot(a, b, preferred_element_type=_F32)


def _dot_bt(a, b):
    return jax.lax.dot_general(a, b, (((1,), (1,)), ((), ())),
                               preferred_element_type=_F32)


def _vq_kernel(f0q_ref, we1_ref, be1_ref, we2_ref, be2_ref, we3_ref, be3_ref,
               cb_ref, wd0_ref, bd0_ref, wt1_ref, bdt1_ref, wt2_ref, bdt2_ref,
               wout_ref, bout_ref,
               f0q_out_ref, commit_ref, metrics_ref,
               xs, hp_s, h2_s, q_s, a_s, b_s, cq_s,
               wp2_s, wp3_s, wpd0_s, wpt1_s, wpt2_s, w6_s,
               mlo_s, mmid_s, mhi_s, cbT_s, cb2_s, counts_scr, acc_scr):
    g = pl.program_id(0)
    ng = pl.num_programs(0)
    T, S, R, NBL = _T, _S, _R, _NBL
    K = 512
    D = 128
    B = 16

    relu = lambda v: jnp.maximum(v, 0.0)

    @pl.when(g == 0)
    def _():
        for s in (xs, hp_s, h2_s, q_s, a_s, b_s, cq_s):
            for k in range(NBL):
                s[k * S:k * S + 8, :] = jnp.zeros_like(s[0:8, :])
            s[R:R + 16, :] = jnp.zeros_like(s[0:16, :])
        def pack(dst, src_ref, H):
            wt = jnp.transpose(src_ref[...])
            wt3 = wt.reshape(128, H, 128)
            for t in range(H):
                dst[t * 128:(t + 1) * 128, :] = wt3[:, t, :].astype(dst.dtype)
        pack(wp2_s, we2_ref, 4)
        pack(wpt1_s, wt1_ref, 4)
        pack(wpt2_s, wt2_ref, 4)
        pack(wp3_s, we3_ref, 3)
        pack(wpd0_s, wd0_ref, 3)
        w4 = jnp.transpose(we1_ref[...])
        z2r = jnp.zeros((2, 128), _F32)
        z4r = jnp.zeros((4, 128), _F32)
        w6_s[...] = jnp.concatenate(
            [jnp.concatenate([w4, z4r], axis=0),
             jnp.concatenate([z2r, w4, z2r], axis=0)], axis=1)
        cb = cb_ref[...]
        cbT = jnp.transpose(cb)
        cb2_s[...] = jnp.sum(cbT * cbT, axis=0, keepdims=True)
        cbT_s[...] = -2.0 * cbT
        u = wout_ref[...].astype(mlo_s.dtype)
        for m in (mlo_s, mmid_s, mhi_s):
            m[...] = jnp.zeros_like(m[...])
        mlo_s[384:512, 0:1] = u[:, 0:1]
        mmid_s[0:128, 0:1] = u[:, 1:2]
        mmid_s[128:256, 0:1] = u[:, 2:3]
        mmid_s[0:128, 1:2] = u[:, 0:1]
        mmid_s[128:256, 1:2] = u[:, 1:2]
        mmid_s[256:384, 1:2] = u[:, 2:3]
        mmid_s[128:256, 2:3] = u[:, 0:1]
        mmid_s[256:384, 2:3] = u[:, 1:2]
        mmid_s[384:512, 2:3] = u[:, 2:3]
        mmid_s[256:384, 3:4] = u[:, 0:1]
        mmid_s[384:512, 3:4] = u[:, 1:2]
        mhi_s[0:128, 3:4] = u[:, 2:3]

    def win(s, d, c0=None, c1=None):
        if c0 is None:
            return s[7 + d:7 + d + R, :]
        return s[7 + d:7 + d + R, c0:c1]

    def scatter_rows(dst, val, c0=None, c1=None):
        for k in range(NBL):
            if c0 is None:
                dst[k * S + 8:k * S + 8 + T, :] = val[k * S:k * S + T, :]
            else:
                dst[k * S + 8:k * S + 8 + T, c0:c1] = val[k * S:k * S + T, :]

    for k in range(NBL):
        xs[k * S + 8:k * S + 8 + T, :] = f0q_ref[k]
    XX = jnp.concatenate(
        [win(xs, 0, 3, 4), win(xs, 1), win(xs, 2, 0, 1)], axis=1)
    be1 = be1_ref[...]
    b2e1 = jnp.concatenate([be1, be1], axis=1)
    Hp = relu(_dot(XX, w6_s[0:6, :]) + b2e1)
    scatter_rows(hp_s, Hp)
    h2 = relu(_dot(win(hp_s, 0, 128, 256), wp2_s[0:128, :])
              + _dot(win(hp_s, 1), wp2_s[128:384, :])
              + _dot(win(hp_s, 2, 0, 128), wp2_s[384:512, :]) + be2_ref[...])
    scatter_rows(h2_s, h2)
    z = (_dot(win(h2_s, 0), wp3_s[0:128, :])
         + _dot(win(h2_s, 1), wp3_s[128:256, :])
         + _dot(win(h2_s, 2), wp3_s[256:384, :]) + be3_ref[...])

    cb = cb_ref[...]
    dist = _dot(z, cbT_s[...]) + cb2_s[...]
    dmin = jnp.min(dist, axis=1, keepdims=True)
    iota = jax.lax.broadcasted_iota(jnp.int32, (R, K), 1)
    codes = jnp.min(jnp.where(dist <= dmin, iota, K), axis=1, keepdims=True)
    oh = (iota == codes).astype(_F32)
    q = _dot(oh, cb)

    counts_part = jnp.sum(oh, axis=0, keepdims=True)
    for k in range(NBL):
        counts_part -= jnp.sum(oh[k * S + T:(k + 1) * S, :], axis=0,
                               keepdims=True)
    rid = jax.lax.broadcasted_iota(jnp.int32, (R, 1), 0)
    valid = jnp.ones((R, 1), jnp.bool_)
    for k in range(NBL):
        valid = valid & ~((rid >= k * S + T) & (rid < (k + 1) * S))
    mask = valid.astype(_F32)
    diff = z - q
    commit_part = jnp.sum(diff * diff * mask).reshape(1, 1)

    @pl.when(g == 0)
    def _():
        counts_scr[...] = counts_part
        acc_scr[...] = commit_part

    @pl.when(g != 0)
    def _():
        counts_scr[...] += counts_part
        acc_scr[...] += commit_part

    bf = lambda v: v.astype(jnp.bfloat16)
    scatter_rows(q_s, bf(q))
    A = relu(_dot(win(q_s, 0), wpd0_s[0:128, :])
             + _dot(win(q_s, 1), wpd0_s[128:256, :])
             + _dot(win(q_s, 2), wpd0_s[256:384, :]) + bd0_ref[...])
    scatter_rows(a_s, bf(A))
    bdt1 = bdt1_ref[...]
    Wt1m = jnp.concatenate([wpt1_s[256:384, :], wpt1_s[128:256, :]], axis=1)
    v8 = _dot(win(a_s, 1), Wt1m)
    ye = relu(_dot(win(a_s, 0), wpt1_s[0:128, :]) + v8[:, 0:128] + bdt1)
    yo = relu(_dot(win(a_s, 2), wpt1_s[384:512, :]) + v8[:, 128:256] + bdt1)
    scatter_rows(b_s, bf(ye), 0, 128)
    scatter_rows(b_s, bf(yo), 128, 256)
    bdt2 = bdt2_ref[...]
    Wp12 = jnp.concatenate([
        jnp.concatenate([wpt2_s[128:256, :], wpt2_s[0:128, :]], axis=1),
        jnp.concatenate([wpt2_s[384:512, :], wpt2_s[256:384, :]], axis=1)],
        axis=0)
    p0 = relu(_dot(win(b_s, 0, 128, 256), wpt2_s[0:128, :])
              + _dot(win(b_s, 1, 0, 128), wpt2_s[256:384, :]) + bdt2)
    p12 = relu(_dot(win(b_s, 1), Wp12)
               + jnp.concatenate([bdt2, bdt2], axis=1))
    p3 = relu(_dot(win(b_s, 1, 128, 256), wpt2_s[128:256, :])
              + _dot(win(b_s, 2, 0, 128), wpt2_s[384:512, :]) + bdt2)
    scatter_rows(cq_s, bf(p0), 0, 128)
    scatter_rows(cq_s, bf(p12), 128, 384)
    scatter_rows(cq_s, bf(p3), 384, 512)
    vf = (_dot(win(cq_s, 0), mlo_s[...])
          + _dot(win(cq_s, 1), mmid_s[...])
          + _dot(win(cq_s, 2), mhi_s[...]) + bout_ref[0, 0])
    for k in range(NBL):
        f0q_out_ref[k] = vf[k * S:k * S + T, :]

    @pl.when(g == ng - 1)
    def _():
        counts = counts_scr[...]
        probs = counts * (1.0 / (B * T))
        ent = -jnp.sum(probs * jnp.log(probs + 1e-8), axis=1, keepdims=True)
        perp = jnp.exp(ent)
        usage = jnp.sum((counts > 0).astype(_F32), axis=1,
                        keepdims=True) * (1.0 / K)
        metrics_ref[...] = jnp.concatenate([perp, usage], axis=1)
        commit_ref[...] = acc_scr[...] * (1.0 / (B * T * D))


def kernel(f0, w_e1, b_e1, w_e2, b_e2, w_e3, b_e3, codebook,
           w_d0, b_d0, w_dt1, b_dt1, w_dt2, b_dt2, w_out, b_out):
    B, _, L = f0.shape
    W = w_e2.shape[0]
    D = w_e3.shape[0]
    K = codebook.shape[0]
    T = L // 4

    f0q = f0.reshape(B, T, 4)
    row = lambda v: v[None]

    args = (f0q, w_e1.reshape(W, 4), row(b_e1), w_e2.reshape(W, 4 * W),
            row(b_e2), w_e3.reshape(D, 3 * W), row(b_e3), codebook,
            w_d0.reshape(W, 3 * D), row(b_d0), w_dt1.reshape(W, 4 * W),
            row(b_dt1), w_dt2.reshape(W, 4 * W), row(b_dt2),
            w_out.reshape(W, 3), b_out.reshape(1, 1))

    const = lambda arr: pl.BlockSpec(arr.shape, lambda g: (0,) * arr.ndim)
    in_specs = [pl.BlockSpec((_NBL, T, 4), lambda g: (g, 0, 0))]
    in_specs += [const(a) for a in args[1:]]

    f0q_out, commit, metrics = pl.pallas_call(
        _vq_kernel,
        grid=(B // _NBL,),
        in_specs=in_specs,
        out_specs=[
            pl.BlockSpec((_NBL, T, 4), lambda g: (g, 0, 0)),
            pl.BlockSpec((1, 1), lambda g: (0, 0)),
            pl.BlockSpec((1, 2), lambda g: (0, 0)),
        ],
        out_shape=(
            jax.ShapeDtypeStruct((B, T, 4), _F32),
            jax.ShapeDtypeStruct((1, 1), _F32),
            jax.ShapeDtypeStruct((1, 2), _F32),
        ),
        scratch_shapes=[
            pltpu.VMEM((_RS, 4), _F32),
            pltpu.VMEM((_RS, 2 * W), _F32),
            pltpu.VMEM((_RS, W), _F32),
            pltpu.VMEM((_RS, D), jnp.bfloat16),
            pltpu.VMEM((_RS, W), jnp.bfloat16),
            pltpu.VMEM((_RS, 2 * W), jnp.bfloat16),
            pltpu.VMEM((_RS, 4 * W), jnp.bfloat16),
            pltpu.VMEM((4 * W, W), _F32),
            pltpu.VMEM((3 * W, W), _F32),
            pltpu.VMEM((3 * W, W), jnp.bfloat16),
            pltpu.VMEM((4 * W, W), jnp.bfloat16),
            pltpu.VMEM((4 * W, W), jnp.bfloat16),
            pltpu.VMEM((8, 2 * W), _F32),
            pltpu.VMEM((4 * W, 4), jnp.bfloat16),
            pltpu.VMEM((4 * W, 4), jnp.bfloat16),
            pltpu.VMEM((4 * W, 4), jnp.bfloat16),
            pltpu.VMEM((W, K), _F32),
            pltpu.VMEM((1, K), _F32),
            pltpu.VMEM((1, K), _F32),
            pltpu.VMEM((1, 1), _F32),
        ],
        compiler_params=pltpu.CompilerParams(
            dimension_semantics=("arbitrary",),
        ),
    )(*args)

    f0_rec = f0q_out.reshape(B, 1, L)
    return (f0_rec, commit[0, 0], metrics[0])


# speedup vs baseline: 1.0085x; 1.0085x over previous
import jax
import jax.numpy as jnp
from jax.experimental import pallas as pl
from jax.experimental.pallas import tpu as pltpu

_F32 = jnp.float32

_NBL = 8
_T = 512
_S = _T + 8
_R = _NBL * _S
_RS = _R + 16


def _dot(a, b):
    return jnp.dot(a, b, preferred_element_type=_F32)


def _dot_bt(a, b):
    return jax.lax.dot_general(a, b, (((1,), (1,)), ((), ())),
                               preferred_element_type=_F32)


def _vq_kernel(f0q_ref, we1_ref, be1_ref, we2_ref, be2_ref, we3_ref, be3_ref,
               cb_ref, wd0_ref, bd0_ref, wt1_ref, bdt1_ref, wt2_ref, bdt2_ref,
               wout_ref, bout_ref,
               f0q_out_ref, commit_ref, metrics_ref,
               xs, hp_s, h2_s, q_s, a_s, b_s, cq_s,
               wp2_s, wp3_s, wpd0_s, wpt1_s, wpt2_s, w6_s,
               mlo_s, mmid_s, mhi_s, cbT_s, cb2_s, counts_scr, acc_scr):
    g = pl.program_id(0)
    ng = pl.num_programs(0)
    T, S, R, NBL = _T, _S, _R, _NBL
    K = 512
    D = 128
    B = 16

    relu = lambda v: jnp.maximum(v, 0.0)

    @pl.when(g == 0)
    def _():
        for s in (xs, hp_s, h2_s, q_s, a_s, b_s, cq_s):
            for k in range(NBL):
                s[k * S:k * S + 8, :] = jnp.zeros_like(s[0:8, :])
            s[R:R + 16, :] = jnp.zeros_like(s[0:16, :])
        def pack(dst, src_ref, H):
            wt = jnp.transpose(src_ref[...])
            wt3 = wt.reshape(128, H, 128)
            for t in range(H):
                dst[t * 128:(t + 1) * 128, :] = wt3[:, t, :]
        pack(wp2_s, we2_ref, 4)
        pack(wpt1_s, wt1_ref, 4)
        pack(wpt2_s, wt2_ref, 4)
        pack(wp3_s, we3_ref, 3)
        pack(wpd0_s, wd0_ref, 3)
        w4 = jnp.transpose(we1_ref[...])
        z2r = jnp.zeros((2, 128), _F32)
        z4r = jnp.zeros((4, 128), _F32)
        w6_s[...] = jnp.concatenate(
            [jnp.concatenate([w4, z4r], axis=0),
             jnp.concatenate([z2r, w4, z2r], axis=0)], axis=1)
        cb = cb_ref[...]
        cbT = jnp.transpose(cb)
        cb2_s[...] = jnp.sum(cbT * cbT, axis=0, keepdims=True)
        cbT_s[...] = -2.0 * cbT
        u = wout_ref[...]
        for m in (mlo_s, mmid_s, mhi_s):
            m[...] = jnp.zeros_like(m[...])
        mlo_s[384:512, 0:1] = u[:, 0:1]
        mmid_s[0:128, 0:1] = u[:, 1:2]
        mmid_s[128:256, 0:1] = u[:, 2:3]
        mmid_s[0:128, 1:2] = u[:, 0:1]
        mmid_s[128:256, 1:2] = u[:, 1:2]
        mmid_s[256:384, 1:2] = u[:, 2:3]
        mmid_s[128:256, 2:3] = u[:, 0:1]
        mmid_s[256:384, 2:3] = u[:, 1:2]
        mmid_s[384:512, 2:3] = u[:, 2:3]
        mmid_s[256:384, 3:4] = u[:, 0:1]
        mmid_s[384:512, 3:4] = u[:, 1:2]
        mhi_s[0:128, 3:4] = u[:, 2:3]

    def win(s, d, c0=None, c1=None):
        if c0 is None:
            return s[7 + d:7 + d + R, :]
        return s[7 + d:7 + d + R, c0:c1]

    def scatter_rows(dst, val, c0=None, c1=None):
        for k in range(NBL):
            if c0 is None:
                dst[k * S + 8:k * S + 8 + T, :] = val[k * S:k * S + T, :]
            else:
                dst[k * S + 8:k * S + 8 + T, c0:c1] = val[k * S:k * S + T, :]

    for k in range(NBL):
        xs[k * S + 8:k * S + 8 + T, :] = f0q_ref[k]
    XX = jnp.concatenate(
        [win(xs, 0, 3, 4), win(xs, 1), win(xs, 2, 0, 1)], axis=1)
    be1 = be1_ref[...]
    b2e1 = jnp.concatenate([be1, be1], axis=1)
    Hp = relu(_dot(XX, w6_s[0:6, :]) + b2e1)
    scatter_rows(hp_s, Hp)
    h2 = relu(_dot(win(hp_s, 0, 128, 256), wp2_s[0:128, :])
              + _dot(win(hp_s, 1), wp2_s[128:384, :])
              + _dot(win(hp_s, 2, 0, 128), wp2_s[384:512, :]) + be2_ref[...])
    scatter_rows(h2_s, h2)
    z = (_dot(win(h2_s, 0), wp3_s[0:128, :])
         + _dot(win(h2_s, 1), wp3_s[128:256, :])
         + _dot(win(h2_s, 2), wp3_s[256:384, :]) + be3_ref[...])

    cb = cb_ref[...]
    dist = _dot(z, cbT_s[...]) + cb2_s[...]
    dmin = jnp.min(dist, axis=1, keepdims=True)
    iota = jax.lax.broadcasted_iota(jnp.int32, (R, K), 1)
    codes = jnp.min(jnp.where(dist <= dmin, iota, K), axis=1, keepdims=True)
    oh = (iota == codes).astype(_F32)
    q = _dot(oh, cb)

    counts_part = jnp.sum(oh, axis=0, keepdims=True)
    for k in range(NBL):
        counts_part -= jnp.sum(oh[k * S + T:(k + 1) * S, :], axis=0,
                               keepdims=True)
    rid = jax.lax.broadcasted_iota(jnp.int32, (R, 1), 0)
    valid = jnp.ones((R, 1), jnp.bool_)
    for k in range(NBL):
        valid = valid & ~((rid >= k * S + T) & (rid < (k + 1) * S))
    mask = valid.astype(_F32)
    diff = z - q
    commit_part = jnp.sum(diff * diff * mask).reshape(1, 1)

    @pl.when(g == 0)
    def _():
        counts_scr[...] = counts_part
        acc_scr[...] = commit_part

    @pl.when(g != 0)
    def _():
        counts_scr[...] += counts_part
        acc_scr[...] += commit_part

    scatter_rows(q_s, q)
    A = relu(_dot(win(q_s, 0), wpd0_s[0:128, :])
             + _dot(win(q_s, 1), wpd0_s[128:256, :])
             + _dot(win(q_s, 2), wpd0_s[256:384, :]) + bd0_ref[...])
    scatter_rows(a_s, A)
    bdt1 = bdt1_ref[...]
    Wt1m = jnp.concatenate([wpt1_s[256:384, :], wpt1_s[128:256, :]], axis=1)
    v8 = _dot(win(a_s, 1), Wt1m)
    ye = relu(_dot(win(a_s, 0), wpt1_s[0:128, :]) + v8[:, 0:128] + bdt1)
    yo = relu(_dot(win(a_s, 2), wpt1_s[384:512, :]) + v8[:, 128:256] + bdt1)
    scatter_rows(b_s, ye, 0, 128)
    scatter_rows(b_s, yo, 128, 256)
    bdt2 = bdt2_ref[...]
    Wp12 = jnp.concatenate([
        jnp.concatenate([wpt2_s[128:256, :], wpt2_s[0:128, :]], axis=1),
        jnp.concatenate([wpt2_s[384:512, :], wpt2_s[256:384, :]], axis=1)],
        axis=0)
    p0 = relu(_dot(win(b_s, 0, 128, 256), wpt2_s[0:128, :])
              + _dot(win(b_s, 1, 0, 128), wpt2_s[256:384, :]) + bdt2)
    p12 = relu(_dot(win(b_s, 1), Wp12)
               + jnp.concatenate([bdt2, bdt2], axis=1))
    p3 = relu(_dot(win(b_s, 1, 128, 256), wpt2_s[128:256, :])
              + _dot(win(b_s, 2, 0, 128), wpt2_s[384:512, :]) + bdt2)
    scatter_rows(cq_s, p0, 0, 128)
    scatter_rows(cq_s, p12, 128, 384)
    scatter_rows(cq_s, p3, 384, 512)
    vf = (_dot(win(cq_s, 0), mlo_s[...])
          + _dot(win(cq_s, 1), mmid_s[...])
          + _dot(win(cq_s, 2), mhi_s[...]) + bout_ref[0, 0])
    for k in range(NBL):
        f0q_out_ref[k] = vf[k * S:k * S + T, :]

    @pl.when(g == ng - 1)
    def _():
        counts = counts_scr[...]
        probs = counts * (1.0 / (B * T))
        ent = -jnp.sum(probs * jnp.log(probs + 1e-8), axis=1, keepdims=True)
        perp = jnp.exp(ent)
        usage = jnp.sum((counts > 0).astype(_F32), axis=1,
                        keepdims=True) * (1.0 / K)
        metrics_ref[...] = jnp.concatenate([perp, usage], axis=1)
        commit_ref[...] = acc_scr[...] * (1.0 / (B * T * D))


def kernel(f0, w_e1, b_e1, w_e2, b_e2, w_e3, b_e3, codebook,
           w_d0, b_d0, w_dt1, b_dt1, w_dt2, b_dt2, w_out, b_out):
    B, _, L = f0.shape
    W = w_e2.shape[0]
    D = w_e3.shape[0]
    K = codebook.shape[0]
    T = L // 4

    f0q = f0.reshape(B, T, 4)
    row = lambda v: v[None]

    args = (f0q, w_e1.reshape(W, 4), row(b_e1), w_e2.reshape(W, 4 * W),
            row(b_e2), w_e3.reshape(D, 3 * W), row(b_e3), codebook,
            w_d0.reshape(W, 3 * D), row(b_d0), w_dt1.reshape(W, 4 * W),
            row(b_dt1), w_dt2.reshape(W, 4 * W), row(b_dt2),
            w_out.reshape(W, 3), b_out.reshape(1, 1))

    const = lambda arr: pl.BlockSpec(arr.shape, lambda g: (0,) * arr.ndim)
    in_specs = [pl.BlockSpec((_NBL, T, 4), lambda g: (g, 0, 0))]
    in_specs += [const(a) for a in args[1:]]

    f0q_out, commit, metrics = pl.pallas_call(
        _vq_kernel,
        grid=(B // _NBL,),
        in_specs=in_specs,
        out_specs=[
            pl.BlockSpec((_NBL, T, 4), lambda g: (g, 0, 0)),
            pl.BlockSpec((1, 1), lambda g: (0, 0)),
            pl.BlockSpec((1, 2), lambda g: (0, 0)),
        ],
        out_shape=(
            jax.ShapeDtypeStruct((B, T, 4), _F32),
            jax.ShapeDtypeStruct((1, 1), _F32),
            jax.ShapeDtypeStruct((1, 2), _F32),
        ),
        scratch_shapes=[
            pltpu.VMEM((_RS, 4), _F32),
            pltpu.VMEM((_RS, 2 * W), _F32),
            pltpu.VMEM((_RS, W), _F32),
            pltpu.VMEM((_RS, D), _F32),
            pltpu.VMEM((_RS, W), _F32),
            pltpu.VMEM((_RS, 2 * W), _F32),
            pltpu.VMEM((_RS, 4 * W), _F32),
            pltpu.VMEM((4 * W, W), _F32),
            pltpu.VMEM((3 * W, W), _F32),
            pltpu.VMEM((3 * W, W), _F32),
            pltpu.VMEM((4 * W, W), _F32),
            pltpu.VMEM((4 * W, W), _F32),
            pltpu.VMEM((8, 2 * W), _F32),
            pltpu.VMEM((4 * W, 4), _F32),
            pltpu.VMEM((4 * W, 4), _F32),
            pltpu.VMEM((4 * W, 4), _F32),
            pltpu.VMEM((W, K), _F32),
            pltpu.VMEM((1, K), _F32),
            pltpu.VMEM((1, K), _F32),
            pltpu.VMEM((1, 1), _F32),
        ],
        compiler_params=pltpu.CompilerParams(
            dimension_semantics=("arbitrary",),
        ),
    )(*args)

    f0_rec = f0q_out.reshape(B, 1, L)
    return (f0_rec, commit[0, 0], metrics[0])
